# baseline (speedup 1.0000x reference)
import functools

import jax
import jax.numpy as jnp
from jax.experimental import pallas as pl
from jax.sharding import Mesh, PartitionSpec as P
from jax.experimental.shard_map import shard_map

BM = 512
D = 2048
E = 2048


def _matmul_kernel(x_ref, w_ref, b_ref, o_ref):
    x = x_ref[...].astype(jnp.bfloat16)
    w = w_ref[...].astype(jnp.bfloat16)
    acc = jax.lax.dot_general(
        x, w,
        dimension_numbers=(((1,), (1,)), ((), ())),
        preferred_element_type=jnp.float32,
    )
    o_ref[...] = acc + b_ref[...]


def _linear_pallas(Xf, W, b2):
    M = Xf.shape[0]
    grid = (M // BM,)
    return pl.pallas_call(
        _matmul_kernel,
        grid=grid,
        in_specs=[
            pl.BlockSpec((BM, D), lambda i: (i, 0)),
            pl.BlockSpec((E, D), lambda i: (0, 0)),
            pl.BlockSpec((1, E), lambda i: (0, 0)),
        ],
        out_specs=pl.BlockSpec((BM, E), lambda i: (i, 0)),
        out_shape=jax.ShapeDtypeStruct((M, E), jnp.float32),
    )(Xf, W, b2)


def kernel(X, W, b):
    B, S, Din = X.shape
    M = B * S
    Xf = X.reshape(M, Din)
    b2 = b.reshape(1, E)

    devs = jax.devices()
    nd = 2 if (len(devs) >= 2 and M % (2 * BM) == 0) else 1
    if nd == 1:
        out = _linear_pallas(Xf, W, b2)
        return out.reshape(1, B, S, E)

    mesh = Mesh(devs[:nd], ("d",))
    run = shard_map(
        _linear_pallas,
        mesh=mesh,
        in_specs=(P("d", None), P(None, None), P(None, None)),
        out_specs=P("d", None),
        check_rep=False,
    )
    out = run(Xf, W, b2)
    return out.reshape(1, B, S, E)


# speedup vs baseline: 1.9503x; 1.9503x over previous
import functools

import jax
import jax.numpy as jnp
from jax.experimental import pallas as pl
from jax.sharding import Mesh, PartitionSpec as P
from jax.experimental.shard_map import shard_map

BM = 512
D = 2048
E = 2048


def _matmul_kernel(x_ref, w_ref, b_ref, o_ref):
    x = x_ref[...].astype(jnp.bfloat16)
    w = w_ref[...].astype(jnp.bfloat16)
    acc = jax.lax.dot_general(
        x, w,
        dimension_numbers=(((1,), (1,)), ((), ())),
        preferred_element_type=jnp.float32,
    )
    o_ref[...] = acc + b_ref[...]


def _linear_pallas(Xf, W, b2):
    M = Xf.shape[0]
    grid = (M // BM,)
    return pl.pallas_call(
        _matmul_kernel,
        grid=grid,
        in_specs=[
            pl.BlockSpec((BM, D), lambda i: (i, 0)),
            pl.BlockSpec((E, D), lambda i: (0, 0)),
            pl.BlockSpec((1, E), lambda i: (0, 0)),
        ],
        out_specs=pl.BlockSpec((BM, E), lambda i: (i, 0)),
        out_shape=jax.ShapeDtypeStruct((M, E), jnp.float32),
    )(Xf, W, b2)


def kernel(X, W, b):
    B, S, Din = X.shape
    M = B * S
    Xf = X.reshape(M, Din)
    b2 = b.reshape(1, E)

    devs = jax.devices()
    nd = 2 if (len(devs) >= 2 and M % (2 * BM) == 0) else 1
    if nd == 1:
        out = _linear_pallas(Xf, W, b2)
        return out.reshape(1, B, S, E)

    mesh = Mesh(devs[:nd], ("d",))
    half = M // nd

    def _shard_fn(xf, w, b2_):
        i = jax.lax.axis_index("d")
        xh = jax.lax.dynamic_slice_in_dim(xf, i * half, half, axis=0)
        return _linear_pallas(xh, w, b2_)

    run = shard_map(
        _shard_fn,
        mesh=mesh,
        in_specs=(P(None, None), P(None, None), P(None, None)),
        out_specs=P("d", None),
        check_rep=False,
    )
    out = run(Xf, W, b2)
    return out.reshape(1, B, S, E)


# speedup vs baseline: 5.9270x; 3.0391x over previous
import jax
import jax.numpy as jnp
from jax.experimental import pallas as pl

BM = 512
D = 2048
E = 2048


def _matmul_kernel(x_ref, w_ref, b_ref, o_ref):
    x = x_ref[...].astype(jnp.bfloat16)
    w = w_ref[...].astype(jnp.bfloat16)
    acc = jax.lax.dot_general(
        x, w,
        dimension_numbers=(((1,), (1,)), ((), ())),
        preferred_element_type=jnp.float32,
    )
    o_ref[...] = acc + b_ref[...]


def kernel(X, W, b):
    B, S, Din = X.shape
    M = B * S
    Xf = X.reshape(M, Din)
    b2 = b.reshape(1, E)
    grid = (M // BM,)
    out = pl.pallas_call(
        _matmul_kernel,
        grid=grid,
        in_specs=[
            pl.BlockSpec((BM, Din), lambda i: (i, 0)),
            pl.BlockSpec((E, Din), lambda i: (0, 0)),
            pl.BlockSpec((1, E), lambda i: (0, 0)),
        ],
        out_specs=pl.BlockSpec((BM, E), lambda i: (i, 0)),
        out_shape=jax.ShapeDtypeStruct((M, E), jnp.float32),
    )(Xf, W, b2)
    return out.reshape(1, B, S, E)


# speedup vs baseline: 5.9788x; 1.0087x over previous
import jax
import jax.numpy as jnp
from jax.experimental import pallas as pl

BM = 1024
D = 2048
E = 2048


def _matmul_kernel(x_ref, w_ref, b_ref, o_ref):
    acc = jax.lax.dot_general(
        x_ref[...], w_ref[...],
        dimension_numbers=(((1,), (1,)), ((), ())),
        preferred_element_type=jnp.float32,
        precision=jax.lax.Precision.DEFAULT,
    )
    o_ref[...] = acc + b_ref[...]


def kernel(X, W, b):
    B, S, Din = X.shape
    M = B * S
    Xf = X.reshape(M, Din)
    b2 = b.reshape(1, E)
    grid = (M // BM,)
    out = pl.pallas_call(
        _matmul_kernel,
        grid=grid,
        in_specs=[
            pl.BlockSpec((BM, Din), lambda i: (i, 0)),
            pl.BlockSpec((E, Din), lambda i: (0, 0)),
            pl.BlockSpec((1, E), lambda i: (0, 0)),
        ],
        out_specs=pl.BlockSpec((BM, E), lambda i: (i, 0)),
        out_shape=jax.ShapeDtypeStruct((M, E), jnp.float32),
    )(Xf, W, b2)
    return out.reshape(1, B, S, E)
